# Initial kernel scaffold; baseline (speedup 1.0000x reference)
import jax
import jax.numpy as jnp
from jax.experimental import pallas as pl


def kernel(xs, table):
    raise NotImplementedError("write your pallas kernel here")


# speedup vs baseline: 1.0226x; 1.0226x over previous
import functools

import jax
import jax.numpy as jnp
from jax import lax
from jax.experimental import pallas as pl
from jax.experimental.pallas import tpu as pltpu
from jax.experimental.pallas import tpu_sc as plsc

_NVEC1 = 1000
_NC = 2
_NS = 16
_NW = _NC * _NS
_LANES = 16
_CHUNK = 128


def kernel(xs, table):
    B = xs.shape[0]
    b_per_w = B // _NW
    n_chunks = b_per_w // _CHUNK

    mesh = plsc.VectorSubcoreMesh(core_axis_name="c", subcore_axis_name="s")

    @functools.partial(
        pl.kernel,
        mesh=mesh,
        out_type=jax.ShapeDtypeStruct((B,), jnp.float32),
        scratch_types=[
            pltpu.VMEM((b_per_w,), jnp.int32),
            pltpu.VMEM((b_per_w,), jnp.int32),
            pltpu.VMEM((n_chunks, _CHUNK), jnp.int32),
            pltpu.VMEM((n_chunks, _CHUNK), jnp.float32),
            pltpu.SemaphoreType.DMA,
        ],
    )
    def _k(x0_hbm, x1_hbm, table_hbm, out_hbm, x0_v, x1_v, idx_v, rows_v, sem):
        wid = lax.axis_index("s") * _NC + lax.axis_index("c")
        base = wid * b_per_w

        pltpu.sync_copy(x0_hbm.at[pl.ds(base, b_per_w)], x0_v)
        pltpu.sync_copy(x1_hbm.at[pl.ds(base, b_per_w)], x1_v)

        per_chunk = _CHUNK // _LANES
        for i in range(b_per_w // _LANES):
            x0 = x0_v[pl.ds(i * _LANES, _LANES)]
            x1 = x1_v[pl.ds(i * _LANES, _LANES)]
            flat = x0 * _NVEC1 + x1
            idx_v[i // per_chunk, pl.ds((i % per_chunk) * _LANES, _LANES)] = flat

        copies = []
        for j in range(n_chunks):
            copies.append(
                pltpu.async_copy(
                    table_hbm.at[idx_v.at[j]],
                    rows_v.at[j],
                    sem,
                )
            )
        for c in copies:
            c.wait()

        for j in range(n_chunks):
            pltpu.sync_copy(
                rows_v.at[j], out_hbm.at[pl.ds(base + j * _CHUNK, _CHUNK)]
            )

    x0 = xs[:, 0]
    x1 = xs[:, 1]
    return _k(x0, x1, table.reshape(-1))
